# baseline (device time: 132897 ns/iter reference)
import jax
import jax.numpy as jnp
from jax import lax
from jax.experimental import pallas as pl
from jax.experimental.pallas import tpu as pltpu

_DeviceIdType = getattr(pl, "DeviceIdType", None) or pltpu.DeviceIdType
_sem_signal = getattr(pl, "semaphore_signal", None) or pltpu.semaphore_signal
_sem_wait = getattr(pl, "semaphore_wait", None) or pltpu.semaphore_wait

M = 2048
D = 2048
F = 8192
D_HALF = D // 2
F_HALF = F // 2
NC = 8
FC = F_HALF // NC
XT = 512
NPB = 4


def kernel(x, dy):

    def body(x_ref, dy_ref, out_ref, x16, x_stage, dy_vmem, p_buf,
             red_stage, rs_recv,
             x_sems, dy_sems, out_sems,
             rs_send_sems, rs_recv_sems, ag_send_sems, ag_recv_sems):
        mx = lax.axis_index("x")
        my = lax.axis_index("y")
        x_nbr = (1 - mx, my)
        y_nbr = (mx, 1 - my)

        barrier = pltpu.get_barrier_semaphore()
        for nbr in (x_nbr, y_nbr):
            _sem_signal(barrier, inc=1, device_id=nbr,
                        device_id_type=_DeviceIdType.MESH)

        def dy_fetch(c):
            cp = pltpu.make_async_copy(
                dy_ref.at[:, pl.ds(mx * F_HALF + c * FC, FC)],
                dy_vmem.at[c % 3],
                dy_sems.at[c % 3],
            )
            cp.start()
            return cp

        dy_cps = [None] * NC
        dy_cps[0] = dy_fetch(0)
        dy_cps[1] = dy_fetch(1)

        n_xt = M // XT
        x_cps = [None] * n_xt
        for i in range(n_xt):
            x_cps[i] = pltpu.make_async_copy(
                x_ref.at[pl.ds(i * XT, XT), :], x_stage.at[i % 2],
                x_sems.at[i % 2],
            )
            x_cps[i].start()
            if i >= 1:
                x_cps[i - 1].wait()
                x16[pl.ds((i - 1) * XT, XT), :] = (
                    x_stage[(i - 1) % 2].astype(jnp.bfloat16)
                )
        x_cps[n_xt - 1].wait()
        x16[pl.ds((n_xt - 1) * XT, XT), :] = (
            x_stage[(n_xt - 1) % 2].astype(jnp.bfloat16)
        )

        rs_ops = [None] * NC
        ag_ops = [None] * NC
        out_cps = [None] * NC

        def finish(c):
            slot = c % NPB
            if c >= NPB:
                out_cps[c - NPB].wait()
                ag_ops[c - NPB].wait_send()
            rs_ops[c].wait_recv()
            red_stage[slot] = (
                p_buf[c % NPB, pl.ds(my * D_HALF, D_HALF), :] + rs_recv[c]
            )
            col = pl.ds(mx * F_HALF + c * FC, FC)
            cp = pltpu.make_async_copy(
                red_stage.at[slot], out_ref.at[:, col], out_sems.at[slot]
            )
            cp.start()
            out_cps[c] = cp
            ag = pltpu.make_async_remote_copy(
                src_ref=red_stage.at[slot],
                dst_ref=out_ref.at[:, col],
                send_sem=ag_send_sems.at[c],
                recv_sem=ag_recv_sems.at[c],
                device_id=x_nbr,
                device_id_type=_DeviceIdType.MESH,
            )
            ag.start()
            ag_ops[c] = ag

        for c in range(NC):
            dy_cps[c].wait()
            b = dy_vmem[c % 3].astype(jnp.bfloat16)
            if c + 2 < NC:
                dy_cps[c + 2] = dy_fetch(c + 2)
            if c >= NPB:
                rs_ops[c - NPB].wait_send()

            def mm_half(row0):
                rows = pl.ds(row0, D_HALF)
                p_buf[c % NPB, rows, :] = lax.dot_general(
                    x16[:, rows], b,
                    dimension_numbers=(((0,), (0,)), ((), ())),
                    preferred_element_type=jnp.float32,
                ).astype(jnp.bfloat16)

            mm_half((1 - my) * D_HALF)
            if c == 0:
                _sem_wait(barrier, 2)
            rs = pltpu.make_async_remote_copy(
                src_ref=p_buf.at[c % NPB, pl.ds((1 - my) * D_HALF, D_HALF), :],
                dst_ref=rs_recv.at[c],
                send_sem=rs_send_sems.at[c],
                recv_sem=rs_recv_sems.at[c],
                device_id=y_nbr,
                device_id_type=_DeviceIdType.MESH,
            )
            rs.start()
            rs_ops[c] = rs
            mm_half(my * D_HALF)
            if c >= 2:
                finish(c - 2)
        finish(NC - 2)
        finish(NC - 1)

        for c in range(NC - NPB, NC):
            rs_ops[c].wait_send()
            out_cps[c].wait()
            ag_ops[c].wait_send()
        for c in range(NC):
            ag_ops[c].wait_recv()

    return pl.pallas_call(
        body,
        out_shape=jax.ShapeDtypeStruct((D_HALF, F), jnp.bfloat16),
        in_specs=[
            pl.BlockSpec(memory_space=pl.ANY),
            pl.BlockSpec(memory_space=pl.ANY),
        ],
        out_specs=pl.BlockSpec(memory_space=pl.ANY),
        scratch_shapes=[
            pltpu.VMEM((M, D), jnp.bfloat16),
            pltpu.VMEM((2, XT, D), jnp.float32),
            pltpu.VMEM((3, M, FC), jnp.float32),
            pltpu.VMEM((NPB, D, FC), jnp.bfloat16),
            pltpu.VMEM((NPB, D_HALF, FC), jnp.bfloat16),
            pltpu.VMEM((NC, D_HALF, FC), jnp.bfloat16),
            pltpu.SemaphoreType.DMA((2,)),
            pltpu.SemaphoreType.DMA((3,)),
            pltpu.SemaphoreType.DMA((NPB,)),
            pltpu.SemaphoreType.DMA((NC,)),
            pltpu.SemaphoreType.DMA((NC,)),
            pltpu.SemaphoreType.DMA((NC,)),
            pltpu.SemaphoreType.DMA((NC,)),
        ],
        compiler_params=pltpu.CompilerParams(
            collective_id=0,
            vmem_limit_bytes=60 * 1024 * 1024,
        ),
    )(x, dy)


# device time: 130857 ns/iter; 1.0156x vs baseline; 1.0156x over previous
import jax
import jax.numpy as jnp
from jax import lax
from jax.experimental import pallas as pl
from jax.experimental.pallas import tpu as pltpu

_DeviceIdType = getattr(pl, "DeviceIdType", None) or pltpu.DeviceIdType
_sem_signal = getattr(pl, "semaphore_signal", None) or pltpu.semaphore_signal
_sem_wait = getattr(pl, "semaphore_wait", None) or pltpu.semaphore_wait

M = 2048
D = 2048
F = 8192
D_HALF = D // 2
F_HALF = F // 2
NC = 16
FC = F_HALF // NC
XT = 512
NPB = 4


def kernel(x, dy):

    def body(x_ref, dy_ref, out_ref, x16, x_stage, dy_vmem, p_buf,
             red_stage, rs_recv, ag_recv,
             x_sems, dy_sems, out_sems, out2_sems,
             rs_send_sems, rs_recv_sems, ag_send_sems, ag_recv_sems):
        mx = lax.axis_index("x")
        my = lax.axis_index("y")
        x_nbr = (1 - mx, my)
        y_nbr = (mx, 1 - my)

        barrier = pltpu.get_barrier_semaphore()
        for nbr in (x_nbr, y_nbr):
            _sem_signal(barrier, inc=1, device_id=nbr,
                        device_id_type=_DeviceIdType.MESH)

        def dy_fetch(c):
            cp = pltpu.make_async_copy(
                dy_ref.at[:, pl.ds(mx * F_HALF + c * FC, FC)],
                dy_vmem.at[c % 3],
                dy_sems.at[c % 3],
            )
            cp.start()
            return cp

        dy_cps = [None] * NC
        dy_cps[0] = dy_fetch(0)
        dy_cps[1] = dy_fetch(1)

        n_xt = M // XT
        x_cps = [None] * n_xt
        for i in range(n_xt):
            x_cps[i] = pltpu.make_async_copy(
                x_ref.at[pl.ds(i * XT, XT), :], x_stage.at[i % 2],
                x_sems.at[i % 2],
            )
            x_cps[i].start()
            if i >= 1:
                x_cps[i - 1].wait()
                x16[pl.ds((i - 1) * XT, XT), :] = (
                    x_stage[(i - 1) % 2].astype(jnp.bfloat16)
                )
        x_cps[n_xt - 1].wait()
        x16[pl.ds((n_xt - 1) * XT, XT), :] = (
            x_stage[(n_xt - 1) % 2].astype(jnp.bfloat16)
        )

        rs_ops = [None] * NC
        ag_ops = [None] * NC
        out_cps = [None] * NC
        out2_cps = [None] * NC

        def finish(c):
            slot = c % NPB
            if c >= NPB:
                out_cps[c - NPB].wait()
                ag_ops[c - NPB].wait_send()
            rs_ops[c].wait_recv()
            red_stage[slot] = (
                p_buf[c % NPB, pl.ds(my * D_HALF, D_HALF), :] + rs_recv[c]
            )
            col = pl.ds(mx * F_HALF + c * FC, FC)
            cp = pltpu.make_async_copy(
                red_stage.at[slot], out_ref.at[:, col], out_sems.at[slot]
            )
            cp.start()
            out_cps[c] = cp
            ag = pltpu.make_async_remote_copy(
                src_ref=red_stage.at[slot],
                dst_ref=ag_recv.at[c],
                send_sem=ag_send_sems.at[c],
                recv_sem=ag_recv_sems.at[c],
                device_id=x_nbr,
                device_id_type=_DeviceIdType.MESH,
            )
            ag.start()
            ag_ops[c] = ag

        def gather_out(c):
            ag_ops[c].wait_recv()
            col = pl.ds((1 - mx) * F_HALF + c * FC, FC)
            cp = pltpu.make_async_copy(
                ag_recv.at[c], out_ref.at[:, col], out2_sems.at[c]
            )
            cp.start()
            out2_cps[c] = cp

        for c in range(NC):
            dy_cps[c].wait()
            b = dy_vmem[c % 3].astype(jnp.bfloat16)
            if c + 2 < NC:
                dy_cps[c + 2] = dy_fetch(c + 2)
            if c >= NPB:
                rs_ops[c - NPB].wait_send()

            def mm_half(row0):
                rows = pl.ds(row0, D_HALF)
                p_buf[c % NPB, rows, :] = lax.dot_general(
                    x16[:, rows], b,
                    dimension_numbers=(((0,), (0,)), ((), ())),
                    preferred_element_type=jnp.float32,
                ).astype(jnp.bfloat16)

            mm_half((1 - my) * D_HALF)
            if c == 0:
                _sem_wait(barrier, 2)
            rs = pltpu.make_async_remote_copy(
                src_ref=p_buf.at[c % NPB, pl.ds((1 - my) * D_HALF, D_HALF), :],
                dst_ref=rs_recv.at[c],
                send_sem=rs_send_sems.at[c],
                recv_sem=rs_recv_sems.at[c],
                device_id=y_nbr,
                device_id_type=_DeviceIdType.MESH,
            )
            rs.start()
            rs_ops[c] = rs
            mm_half(my * D_HALF)
            if c >= 2:
                finish(c - 2)
            if c >= 4:
                gather_out(c - 4)
        finish(NC - 2)
        finish(NC - 1)
        for c in range(NC - 4, NC):
            gather_out(c)

        for c in range(NC - NPB, NC):
            rs_ops[c].wait_send()
            out_cps[c].wait()
            ag_ops[c].wait_send()
        for c in range(NC):
            out2_cps[c].wait()

    return pl.pallas_call(
        body,
        out_shape=jax.ShapeDtypeStruct((D_HALF, F), jnp.bfloat16),
        in_specs=[
            pl.BlockSpec(memory_space=pl.ANY),
            pl.BlockSpec(memory_space=pl.ANY),
        ],
        out_specs=pl.BlockSpec(memory_space=pl.ANY),
        scratch_shapes=[
            pltpu.VMEM((M, D), jnp.bfloat16),
            pltpu.VMEM((2, XT, D), jnp.float32),
            pltpu.VMEM((3, M, FC), jnp.float32),
            pltpu.VMEM((NPB, D, FC), jnp.bfloat16),
            pltpu.VMEM((NPB, D_HALF, FC), jnp.bfloat16),
            pltpu.VMEM((NC, D_HALF, FC), jnp.bfloat16),
            pltpu.VMEM((NC, D_HALF, FC), jnp.bfloat16),
            pltpu.SemaphoreType.DMA((2,)),
            pltpu.SemaphoreType.DMA((3,)),
            pltpu.SemaphoreType.DMA((NPB,)),
            pltpu.SemaphoreType.DMA((NC,)),
            pltpu.SemaphoreType.DMA((NC,)),
            pltpu.SemaphoreType.DMA((NC,)),
            pltpu.SemaphoreType.DMA((NC,)),
            pltpu.SemaphoreType.DMA((NC,)),
        ],
        compiler_params=pltpu.CompilerParams(
            collective_id=0,
            vmem_limit_bytes=60 * 1024 * 1024,
        ),
    )(x, dy)
